# Initial kernel scaffold; baseline (speedup 1.0000x reference)
import jax
import jax.numpy as jnp
from jax.experimental import pallas as pl


def kernel(x, edge_index, W_enc, b_enc, Wc1, bc1, Wc2, bc2, Wc3, bc3, Wd1, bd1, Wd2, bd2, Wi1, bi1, Wi2, bi2):
    raise NotImplementedError("write your pallas kernel here")


# speedup vs baseline: 2.8642x; 2.8642x over previous
import jax
import jax.numpy as jnp
from jax.experimental import pallas as pl


def kernel(x, edge_index, W_enc, b_enc, Wc1, bc1, Wc2, bc2, Wc3, bc3, Wd1, bd1, Wd2, bd2, Wi1, bi1, Wi2, bi2):
    n = x.shape[0]
    src = edge_index[0]
    dst = edge_index[1]
    deg = jnp.zeros((n,), jnp.float32).at[dst].add(1.0) + 1.0
    dinv = jax.lax.rsqrt(deg)

    h = jax.nn.relu(x @ W_enc.T + b_enc)
    for (W, b) in ((Wc1, bc1), (Wc2, bc2), (Wc3, bc3)):
        xs = dinv[:, None] * (h @ W.T)
        agg = jnp.zeros_like(xs).at[dst].add(xs[src])
        h = jax.nn.relu(dinv[:, None] * (agg + xs) + b)
    demand = jax.nn.relu(h @ Wd1.T + bd1) @ Wd2.T + bd2
    inventory = jax.nn.relu(h @ Wi1.T + bi1) @ Wi2.T + bi2
    return (demand, inventory)


# speedup vs baseline: 6.6647x; 2.3269x over previous
import functools

import jax
import jax.numpy as jnp
from jax import lax
from jax.experimental import pallas as pl
from jax.experimental.pallas import tpu as pltpu
from jax.experimental.pallas import tpu_sc as plsc

_N = 50000
_H = 64
_NPAD = 50176
_SPS = _NPAD // 16
_EPAD = 802816
_ROWS = _EPAD // 128
_RPS = _ROWS // 16
_RPS_DEG = _ROWS // 32
_BLK = 1000
_G = _N // _BLK


def _mesh():
    return plsc.VectorSubcoreMesh(core_axis_name="c", subcore_axis_name="s")


_SC_PARAMS = pltpu.CompilerParams(use_tc_tiling_on_sc=False)


def _sc_deg(dst2):
    @functools.partial(
        pl.kernel, mesh=_mesh(),
        out_type=jax.ShapeDtypeStruct((2 * _NPAD, 16), jnp.float32),
        scratch_types=[
            pltpu.VMEM((64, 16), jnp.float32),
            pltpu.VMEM((128, 16), jnp.float32),
            pltpu.VMEM((4, 128), jnp.int32),
            pltpu.VMEM_SHARED((_NPAD, 16), jnp.float32),
        ],
        compiler_params=_SC_PARAMS)
    def k(dst_hbm, out_hbm, zbuf, ones_v, didx, dacc):
        c = lax.axis_index("c")
        s = lax.axis_index("s")

        @pl.loop(0, 64)
        def _(i):
            zbuf[i, :] = jnp.zeros((16,), jnp.float32)

        @pl.loop(0, 128)
        def _(i):
            ones_v[i, :] = jnp.ones((16,), jnp.float32)

        @pl.loop(0, _SPS, step=64)
        def _(t):
            pltpu.sync_copy(zbuf, dacc.at[pl.ds(s * _SPS + t, 64)])

        plsc.subcore_barrier()

        @pl.loop(0, _RPS_DEG, step=4)
        def _(r):
            row0 = (c * 16 + s) * _RPS_DEG + r
            pltpu.sync_copy(dst_hbm.at[pl.ds(row0, 4)], didx)
            for j in range(4):
                pltpu.sync_copy(ones_v, dacc.at[didx.at[j]], add=True)

        plsc.subcore_barrier()
        pltpu.sync_copy(dacc.at[pl.ds(s * _SPS, _SPS)],
                        out_hbm.at[pl.ds(c * _NPAD + s * _SPS, _SPS)])

    return k(dst2)


def _sc_conv(xsf, src2, dst2):
    @functools.partial(
        pl.kernel, mesh=_mesh(),
        out_type=jax.ShapeDtypeStruct((2 * _NPAD, 32), jnp.float32),
        scratch_types=[
            pltpu.VMEM((64, 32), jnp.float32),
            pltpu.VMEM((4, 128), jnp.int32),
            pltpu.VMEM((4, 128), jnp.int32),
            pltpu.VMEM((128, 128), jnp.float32),
            pltpu.VMEM((128, 32), jnp.float32),
            pltpu.VMEM_SHARED((_NPAD, 32), jnp.float32),
        ],
        compiler_params=_SC_PARAMS)
    def k(xs_hbm, src_hbm, dst_hbm, out_hbm, zbuf, sidx, didx, rows, rows32,
          acc):
        c = lax.axis_index("c")
        s = lax.axis_index("s")

        @pl.loop(0, 64)
        def _(i):
            zbuf[i, pl.ds(0, 16)] = jnp.zeros((16,), jnp.float32)
            zbuf[i, pl.ds(16, 16)] = jnp.zeros((16,), jnp.float32)

        @pl.loop(0, _SPS, step=64)
        def _(t):
            pltpu.sync_copy(zbuf, acc.at[pl.ds(s * _SPS + t, 64)])

        plsc.subcore_barrier()

        def edge_loop(off):
            @pl.loop(0, _RPS, step=4)
            def _(r):
                row0 = s * _RPS + r
                pltpu.sync_copy(src_hbm.at[pl.ds(row0, 4)], sidx)
                pltpu.sync_copy(dst_hbm.at[pl.ds(row0, 4)], didx)
                for j in range(4):
                    pltpu.sync_copy(xs_hbm.at[sidx.at[j]], rows)

                    @pl.loop(0, 128)
                    def _(i):
                        rows32[i, pl.ds(0, 16)] = rows[i, pl.ds(off, 16)]
                        rows32[i, pl.ds(16, 16)] = rows[i, pl.ds(off + 16, 16)]

                    pltpu.sync_copy(rows32, acc.at[didx.at[j]], add=True)

        @pl.when(c == 0)
        def _():
            edge_loop(0)

        @pl.when(c == 1)
        def _():
            edge_loop(32)

        plsc.subcore_barrier()
        pltpu.sync_copy(acc.at[pl.ds(s * _SPS, _SPS)],
                        out_hbm.at[pl.ds(c * _NPAD + s * _SPS, _SPS)])

    return k(xsf, src2, dst2)


_DOT = dict(preferred_element_type=jnp.float32, precision=lax.Precision.HIGHEST)


def _dinv_of(degp_blk):
    d16 = lax.rsqrt(degp_blk[0] + degp_blk[1] + 1.0)
    return jnp.concatenate([d16, d16, d16, d16], axis=1)


def _enc_body(x_ref, we_ref, be_ref, wc1_ref, o_ref):
    h = jnp.maximum(
        lax.dot_general(x_ref[...], we_ref[...], (((1,), (1,)), ((), ())), **_DOT)
        + be_ref[...], 0.0)
    o_ref[...] = lax.dot_general(h, wc1_ref[...], (((1,), (1,)), ((), ())), **_DOT)


def _tc_encode(x, W_enc, b_enc, Wc1):
    return pl.pallas_call(
        _enc_body,
        grid=(_G,),
        in_specs=[
            pl.BlockSpec((_BLK, 128), lambda i: (i, 0)),
            pl.BlockSpec((64, 128), lambda i: (0, 0)),
            pl.BlockSpec((1, 64), lambda i: (0, 0)),
            pl.BlockSpec((64, 64), lambda i: (0, 0)),
        ],
        out_specs=pl.BlockSpec((_BLK, 64), lambda i: (i, 0)),
        out_shape=jax.ShapeDtypeStruct((_N, 64), jnp.float32),
    )(x, W_enc, b_enc.reshape(1, 64), Wc1)


def _pad128(xs):
    return jnp.concatenate([xs, jnp.zeros((_BLK, 64), jnp.float32)], axis=1)


def _prep_body(degp_ref, xw_ref, o_ref):
    dinv = _dinv_of(degp_ref[...])
    o_ref[...] = _pad128(xw_ref[...] * dinv)


def _tc_prep(degp, xw1):
    return pl.pallas_call(
        _prep_body,
        grid=(_G,),
        in_specs=[
            pl.BlockSpec((2, _BLK, 16), lambda i: (0, i, 0)),
            pl.BlockSpec((_BLK, 64), lambda i: (i, 0)),
        ],
        out_specs=pl.BlockSpec((_BLK, 128), lambda i: (i, 0)),
        out_shape=jax.ShapeDtypeStruct((_N, 128), jnp.float32),
    )(degp, xw1)


def _mid_body(agg_ref, xs_ref, degp_ref, b_ref, w_ref, o_ref):
    dinv = _dinv_of(degp_ref[...])
    pre = jnp.concatenate(
        [agg_ref[0] + xs_ref[:, :32], agg_ref[1] + xs_ref[:, 32:64]], axis=1)
    h = jnp.maximum(pre * dinv + b_ref[...], 0.0)
    xw = lax.dot_general(h, w_ref[...], (((1,), (1,)), ((), ())), **_DOT)
    o_ref[...] = _pad128(xw * dinv)


def _tc_mid(agg, xs, degp, b_prev, W_next):
    return pl.pallas_call(
        _mid_body,
        grid=(_G,),
        in_specs=[
            pl.BlockSpec((2, _BLK, 32), lambda i: (0, i, 0)),
            pl.BlockSpec((_BLK, 128), lambda i: (i, 0)),
            pl.BlockSpec((2, _BLK, 16), lambda i: (0, i, 0)),
            pl.BlockSpec((1, 64), lambda i: (0, 0)),
            pl.BlockSpec((64, 64), lambda i: (0, 0)),
        ],
        out_specs=pl.BlockSpec((_BLK, 128), lambda i: (i, 0)),
        out_shape=jax.ShapeDtypeStruct((_N, 128), jnp.float32),
    )(agg, xs, degp, b_prev.reshape(1, 64), W_next)


def _heads_body(agg_ref, xs_ref, degp_ref, b_ref, wd1_ref, bd1_ref, wd2_ref,
                bd2_ref, wi1_ref, bi1_ref, wi2_ref, bi2_ref, od_ref, oi_ref):
    dinv = _dinv_of(degp_ref[...])
    pre = jnp.concatenate(
        [agg_ref[0] + xs_ref[:, :32], agg_ref[1] + xs_ref[:, 32:64]], axis=1)
    h = jnp.maximum(pre * dinv + b_ref[...], 0.0)
    hd = jnp.maximum(
        lax.dot_general(h, wd1_ref[...], (((1,), (1,)), ((), ())), **_DOT)
        + bd1_ref[...], 0.0)
    od_ref[...] = lax.dot_general(hd, wd2_ref[...], (((1,), (1,)), ((), ())),
                                  **_DOT) + bd2_ref[...]
    hi = jnp.maximum(
        lax.dot_general(h, wi1_ref[...], (((1,), (1,)), ((), ())), **_DOT)
        + bi1_ref[...], 0.0)
    oi_ref[...] = lax.dot_general(hi, wi2_ref[...], (((1,), (1,)), ((), ())),
                                  **_DOT) + bi2_ref[...]


def _pad_head(W2, b2):
    W2p = jnp.concatenate([W2, jnp.zeros((127, 32), jnp.float32)], axis=0)
    b2p = jnp.broadcast_to(b2.reshape(1, 1), (1, 128))
    return W2p, b2p


def _tc_heads(agg, xs, degp, bc3, Wd1, bd1, Wd2, bd2, Wi1, bi1, Wi2, bi2):
    full = lambda a, b: pl.BlockSpec((a, b), lambda i: (0, 0))
    return pl.pallas_call(
        _heads_body,
        grid=(_G,),
        in_specs=[
            pl.BlockSpec((2, _BLK, 32), lambda i: (0, i, 0)),
            pl.BlockSpec((_BLK, 128), lambda i: (i, 0)),
            pl.BlockSpec((2, _BLK, 16), lambda i: (0, i, 0)),
            full(1, 64),
            full(32, 64), full(1, 32), full(128, 32), full(1, 128),
            full(32, 64), full(1, 32), full(128, 32), full(1, 128),
        ],
        out_specs=[
            pl.BlockSpec((_BLK, 128), lambda i: (i, 0)),
            pl.BlockSpec((_BLK, 128), lambda i: (i, 0)),
        ],
        out_shape=[
            jax.ShapeDtypeStruct((_N, 128), jnp.float32),
            jax.ShapeDtypeStruct((_N, 128), jnp.float32),
        ],
    )(agg, xs, degp, bc3.reshape(1, 64),
      Wd1, bd1.reshape(1, 32), *_pad_head(Wd2, bd2),
      Wi1, bi1.reshape(1, 32), *_pad_head(Wi2, bi2))


def kernel(x, edge_index, W_enc, b_enc, Wc1, bc1, Wc2, bc2, Wc3, bc3,
           Wd1, bd1, Wd2, bd2, Wi1, bi1, Wi2, bi2):
    e = edge_index.shape[1]
    pad = _EPAD - e
    src2 = jnp.concatenate(
        [edge_index[0], jnp.zeros((pad,), jnp.int32)]).reshape(_ROWS, 128)
    dst2 = jnp.concatenate(
        [edge_index[1], jnp.full((pad,), _N, jnp.int32)]).reshape(_ROWS, 128)
    degp = _sc_deg(dst2).reshape(2, _NPAD, 16)
    xw1 = _tc_encode(x, W_enc, b_enc, Wc1)
    xs1 = _tc_prep(degp, xw1)
    agg1 = _sc_conv(xs1, src2, dst2).reshape(2, _NPAD, 32)
    xs2 = _tc_mid(agg1, xs1, degp, bc1, Wc2)
    agg2 = _sc_conv(xs2, src2, dst2).reshape(2, _NPAD, 32)
    xs3 = _tc_mid(agg2, xs2, degp, bc2, Wc3)
    agg3 = _sc_conv(xs3, src2, dst2).reshape(2, _NPAD, 32)
    demand, inventory = _tc_heads(agg3, xs3, degp, bc3,
                                  Wd1, bd1, Wd2, bd2, Wi1, bi1, Wi2, bi2)
    return (demand[:, :1], inventory[:, :1])


# speedup vs baseline: 9.3269x; 1.3994x over previous
import functools

import jax
import jax.numpy as jnp
from jax import lax
from jax.experimental import pallas as pl
from jax.experimental.pallas import tpu as pltpu
from jax.experimental.pallas import tpu_sc as plsc

_N = 50000
_H = 64
_NPAD = 50176
_SPS = _NPAD // 16
_EPAD = 802816
_ROWS = _EPAD // 128
_ROWS64 = _EPAD // 64
_RPS = _ROWS64 // 16
_RPS_DEG = _ROWS // 32
_BLK = 1000
_G = _N // _BLK


def _mesh():
    return plsc.VectorSubcoreMesh(core_axis_name="c", subcore_axis_name="s")


_SC_PARAMS = pltpu.CompilerParams(use_tc_tiling_on_sc=False)


def _sc_deg(dst2):
    @functools.partial(
        pl.kernel, mesh=_mesh(),
        out_type=jax.ShapeDtypeStruct((2 * _NPAD, 16), jnp.float32),
        scratch_types=[
            pltpu.VMEM((64, 16), jnp.float32),
            pltpu.VMEM((128, 16), jnp.float32),
            pltpu.VMEM((4, 128), jnp.int32),
            pltpu.VMEM_SHARED((_NPAD, 16), jnp.float32),
        ],
        compiler_params=_SC_PARAMS)
    def k(dst_hbm, out_hbm, zbuf, ones_v, didx, dacc):
        c = lax.axis_index("c")
        s = lax.axis_index("s")

        @pl.loop(0, 64)
        def _(i):
            zbuf[i, :] = jnp.zeros((16,), jnp.float32)

        @pl.loop(0, 128)
        def _(i):
            ones_v[i, :] = jnp.ones((16,), jnp.float32)

        @pl.loop(0, _SPS, step=64)
        def _(t):
            pltpu.sync_copy(zbuf, dacc.at[pl.ds(s * _SPS + t, 64)])

        plsc.subcore_barrier()

        @pl.loop(0, _RPS_DEG, step=4)
        def _(r):
            row0 = (c * 16 + s) * _RPS_DEG + r
            pltpu.sync_copy(dst_hbm.at[pl.ds(row0, 4)], didx)
            for j in range(4):
                pltpu.sync_copy(ones_v, dacc.at[didx.at[j]], add=True)

        plsc.subcore_barrier()
        pltpu.sync_copy(dacc.at[pl.ds(s * _SPS, _SPS)],
                        out_hbm.at[pl.ds(c * _NPAD + s * _SPS, _SPS)])

    return k(dst2)


def _sc_conv(xsf, src2, dst2):
    @functools.partial(
        pl.kernel, mesh=_mesh(),
        out_type=jax.ShapeDtypeStruct((2 * _NPAD, 32), jnp.float32),
        scratch_types=[
            pltpu.VMEM((64, 32), jnp.float32),
            pltpu.VMEM((1, 64), jnp.int32),
            pltpu.VMEM((1, 64), jnp.int32),
            pltpu.VMEM((1, 64), jnp.int32),
            pltpu.VMEM((1, 64), jnp.int32),
            pltpu.VMEM((64, 128), jnp.float32),
            pltpu.VMEM((64, 128), jnp.float32),
            pltpu.VMEM((64, 32), jnp.float32),
            pltpu.VMEM((64, 32), jnp.float32),
            pltpu.VMEM_SHARED((_NPAD, 32), jnp.float32),
            pltpu.SemaphoreType.DMA,
            pltpu.SemaphoreType.DMA,
            pltpu.SemaphoreType.DMA,
            pltpu.SemaphoreType.DMA,
            pltpu.SemaphoreType.DMA,
            pltpu.SemaphoreType.DMA,
            pltpu.SemaphoreType.DMA,
            pltpu.SemaphoreType.DMA,
        ],
        compiler_params=_SC_PARAMS)
    def k(xs_hbm, src_hbm, dst_hbm, out_hbm, zbuf, sidxA, didxA, sidxB, didxB,
          rowsA, rowsB, r32A, r32B, acc, gA, gB, sA, sB, iA, iB, jA, jB):
        c = lax.axis_index("c")
        s = lax.axis_index("s")
        off = c * 32

        @pl.loop(0, 64)
        def _(i):
            zbuf[i, pl.ds(0, 16)] = jnp.zeros((16,), jnp.float32)
            zbuf[i, pl.ds(16, 16)] = jnp.zeros((16,), jnp.float32)

        @pl.loop(0, _SPS, step=64)
        def _(t):
            pltpu.sync_copy(zbuf, acc.at[pl.ds(s * _SPS + t, 64)])

        plsc.subcore_barrier()

        base = s * _RPS

        def phase(t, sidx, didx, rows, r32, gsem, ssem, isem, jsem):
            pltpu.make_async_copy(xs_hbm.at[sidx.at[0]], rows, gsem).wait()

            @pl.when(t + 2 < _RPS)
            def _():
                pltpu.async_copy(src_hbm.at[pl.ds(base + t + 2, 1)], sidx,
                                 isem)

            @pl.when(t > 1)
            def _():
                pltpu.make_async_copy(r32, acc.at[didx.at[0]], ssem).wait()

            pltpu.async_copy(dst_hbm.at[pl.ds(base + t, 1)], didx, jsem)

            @pl.loop(0, 64)
            def _(i):
                r32[i, pl.ds(0, 16)] = rows[i, pl.ds(off, 16)]
                r32[i, pl.ds(16, 16)] = rows[i, pl.ds(off + 16, 16)]

            pltpu.make_async_copy(dst_hbm.at[pl.ds(base + t, 1)], didx,
                                  jsem).wait()
            pltpu.async_copy(r32, acc.at[didx.at[0]], ssem, add=True)

            @pl.when(t + 2 < _RPS)
            def _():
                pltpu.make_async_copy(src_hbm.at[pl.ds(base, 1)], sidx,
                                      isem).wait()
                pltpu.async_copy(xs_hbm.at[sidx.at[0]], rows, gsem)

        pltpu.sync_copy(src_hbm.at[pl.ds(base, 1)], sidxA)
        pltpu.sync_copy(src_hbm.at[pl.ds(base + 1, 1)], sidxB)
        pltpu.async_copy(xs_hbm.at[sidxA.at[0]], rowsA, gA)
        pltpu.async_copy(xs_hbm.at[sidxB.at[0]], rowsB, gB)

        @pl.loop(0, _RPS, step=2)
        def _(t):
            phase(t, sidxA, didxA, rowsA, r32A, gA, sA, iA, jA)
            phase(t + 1, sidxB, didxB, rowsB, r32B, gB, sB, iB, jB)

        pltpu.make_async_copy(r32A, acc.at[didxA.at[0]], sA).wait()
        pltpu.make_async_copy(r32B, acc.at[didxB.at[0]], sB).wait()

        plsc.subcore_barrier()
        pltpu.sync_copy(acc.at[pl.ds(s * _SPS, _SPS)],
                        out_hbm.at[pl.ds(c * _NPAD + s * _SPS, _SPS)])

    return k(xsf, src2, dst2)


_DOT = dict(preferred_element_type=jnp.float32, precision=lax.Precision.HIGHEST)


def _dinv_of(degp_blk):
    d16 = lax.rsqrt(degp_blk[0] + degp_blk[1] + 1.0)
    return jnp.concatenate([d16, d16, d16, d16], axis=1)


def _enc_body(x_ref, we_ref, be_ref, wc1_ref, o_ref):
    h = jnp.maximum(
        lax.dot_general(x_ref[...], we_ref[...], (((1,), (1,)), ((), ())), **_DOT)
        + be_ref[...], 0.0)
    o_ref[...] = lax.dot_general(h, wc1_ref[...], (((1,), (1,)), ((), ())), **_DOT)


def _tc_encode(x, W_enc, b_enc, Wc1):
    return pl.pallas_call(
        _enc_body,
        grid=(_G,),
        in_specs=[
            pl.BlockSpec((_BLK, 128), lambda i: (i, 0)),
            pl.BlockSpec((64, 128), lambda i: (0, 0)),
            pl.BlockSpec((1, 64), lambda i: (0, 0)),
            pl.BlockSpec((64, 64), lambda i: (0, 0)),
        ],
        out_specs=pl.BlockSpec((_BLK, 64), lambda i: (i, 0)),
        out_shape=jax.ShapeDtypeStruct((_N, 64), jnp.float32),
    )(x, W_enc, b_enc.reshape(1, 64), Wc1)


def _pad128(xs):
    return jnp.concatenate([xs, jnp.zeros((_BLK, 64), jnp.float32)], axis=1)


def _prep_body(degp_ref, xw_ref, o_ref):
    dinv = _dinv_of(degp_ref[...])
    o_ref[...] = _pad128(xw_ref[...] * dinv)


def _tc_prep(degp, xw1):
    return pl.pallas_call(
        _prep_body,
        grid=(_G,),
        in_specs=[
            pl.BlockSpec((2, _BLK, 16), lambda i: (0, i, 0)),
            pl.BlockSpec((_BLK, 64), lambda i: (i, 0)),
        ],
        out_specs=pl.BlockSpec((_BLK, 128), lambda i: (i, 0)),
        out_shape=jax.ShapeDtypeStruct((_N, 128), jnp.float32),
    )(degp, xw1)


def _mid_body(agg_ref, xs_ref, degp_ref, b_ref, w_ref, o_ref):
    dinv = _dinv_of(degp_ref[...])
    pre = jnp.concatenate(
        [agg_ref[0] + xs_ref[:, :32], agg_ref[1] + xs_ref[:, 32:64]], axis=1)
    h = jnp.maximum(pre * dinv + b_ref[...], 0.0)
    xw = lax.dot_general(h, w_ref[...], (((1,), (1,)), ((), ())), **_DOT)
    o_ref[...] = _pad128(xw * dinv)


def _tc_mid(agg, xs, degp, b_prev, W_next):
    return pl.pallas_call(
        _mid_body,
        grid=(_G,),
        in_specs=[
            pl.BlockSpec((2, _BLK, 32), lambda i: (0, i, 0)),
            pl.BlockSpec((_BLK, 128), lambda i: (i, 0)),
            pl.BlockSpec((2, _BLK, 16), lambda i: (0, i, 0)),
            pl.BlockSpec((1, 64), lambda i: (0, 0)),
            pl.BlockSpec((64, 64), lambda i: (0, 0)),
        ],
        out_specs=pl.BlockSpec((_BLK, 128), lambda i: (i, 0)),
        out_shape=jax.ShapeDtypeStruct((_N, 128), jnp.float32),
    )(agg, xs, degp, b_prev.reshape(1, 64), W_next)


def _heads_body(agg_ref, xs_ref, degp_ref, b_ref, wd1_ref, bd1_ref, wd2_ref,
                bd2_ref, wi1_ref, bi1_ref, wi2_ref, bi2_ref, od_ref, oi_ref):
    dinv = _dinv_of(degp_ref[...])
    pre = jnp.concatenate(
        [agg_ref[0] + xs_ref[:, :32], agg_ref[1] + xs_ref[:, 32:64]], axis=1)
    h = jnp.maximum(pre * dinv + b_ref[...], 0.0)
    hd = jnp.maximum(
        lax.dot_general(h, wd1_ref[...], (((1,), (1,)), ((), ())), **_DOT)
        + bd1_ref[...], 0.0)
    od_ref[...] = lax.dot_general(hd, wd2_ref[...], (((1,), (1,)), ((), ())),
                                  **_DOT) + bd2_ref[...]
    hi = jnp.maximum(
        lax.dot_general(h, wi1_ref[...], (((1,), (1,)), ((), ())), **_DOT)
        + bi1_ref[...], 0.0)
    oi_ref[...] = lax.dot_general(hi, wi2_ref[...], (((1,), (1,)), ((), ())),
                                  **_DOT) + bi2_ref[...]


def _pad_head(W2, b2):
    W2p = jnp.concatenate([W2, jnp.zeros((127, 32), jnp.float32)], axis=0)
    b2p = jnp.broadcast_to(b2.reshape(1, 1), (1, 128))
    return W2p, b2p


def _tc_heads(agg, xs, degp, bc3, Wd1, bd1, Wd2, bd2, Wi1, bi1, Wi2, bi2):
    full = lambda a, b: pl.BlockSpec((a, b), lambda i: (0, 0))
    return pl.pallas_call(
        _heads_body,
        grid=(_G,),
        in_specs=[
            pl.BlockSpec((2, _BLK, 32), lambda i: (0, i, 0)),
            pl.BlockSpec((_BLK, 128), lambda i: (i, 0)),
            pl.BlockSpec((2, _BLK, 16), lambda i: (0, i, 0)),
            full(1, 64),
            full(32, 64), full(1, 32), full(128, 32), full(1, 128),
            full(32, 64), full(1, 32), full(128, 32), full(1, 128),
        ],
        out_specs=[
            pl.BlockSpec((_BLK, 128), lambda i: (i, 0)),
            pl.BlockSpec((_BLK, 128), lambda i: (i, 0)),
        ],
        out_shape=[
            jax.ShapeDtypeStruct((_N, 128), jnp.float32),
            jax.ShapeDtypeStruct((_N, 128), jnp.float32),
        ],
    )(agg, xs, degp, bc3.reshape(1, 64),
      Wd1, bd1.reshape(1, 32), *_pad_head(Wd2, bd2),
      Wi1, bi1.reshape(1, 32), *_pad_head(Wi2, bi2))


def kernel(x, edge_index, W_enc, b_enc, Wc1, bc1, Wc2, bc2, Wc3, bc3,
           Wd1, bd1, Wd2, bd2, Wi1, bi1, Wi2, bi2):
    e = edge_index.shape[1]
    pad = _EPAD - e
    srcf = jnp.concatenate([edge_index[0], jnp.zeros((pad,), jnp.int32)])
    dstf = jnp.concatenate([edge_index[1], jnp.full((pad,), _N, jnp.int32)])
    src2 = srcf.reshape(_ROWS64, 64)
    dst2 = dstf.reshape(_ROWS64, 64)
    degp = _sc_deg(dstf.reshape(_ROWS, 128)).reshape(2, _NPAD, 16)
    xw1 = _tc_encode(x, W_enc, b_enc, Wc1)
    xs1 = _tc_prep(degp, xw1)
    agg1 = _sc_conv(xs1, src2, dst2).reshape(2, _NPAD, 32)
    xs2 = _tc_mid(agg1, xs1, degp, bc1, Wc2)
    agg2 = _sc_conv(xs2, src2, dst2).reshape(2, _NPAD, 32)
    xs3 = _tc_mid(agg2, xs2, degp, bc2, Wc3)
    agg3 = _sc_conv(xs3, src2, dst2).reshape(2, _NPAD, 32)
    demand, inventory = _tc_heads(agg3, xs3, degp, bc3,
                                  Wd1, bd1, Wd2, bd2, Wi1, bi1, Wi2, bi2)
    return (demand[:, :1], inventory[:, :1])


# speedup vs baseline: 9.3825x; 1.0060x over previous
import functools

import jax
import jax.numpy as jnp
from jax import lax
from jax.experimental import pallas as pl
from jax.experimental.pallas import tpu as pltpu
from jax.experimental.pallas import tpu_sc as plsc

_N = 50000
_H = 64
_NPAD = 50176
_SPS = _NPAD // 16
_EPAD = 802816
_ROWS = _EPAD // 128
_ROWS64 = _EPAD // 64
_RPS = _ROWS64 // 16
_RPS_DEG = _ROWS // 32
_BLK = 1000
_G = _N // _BLK


def _mesh():
    return plsc.VectorSubcoreMesh(core_axis_name="c", subcore_axis_name="s")


_SC_PARAMS = pltpu.CompilerParams(use_tc_tiling_on_sc=False)


def _sc_deg(dst2):
    @functools.partial(
        pl.kernel, mesh=_mesh(),
        out_type=jax.ShapeDtypeStruct((2 * _NPAD, 16), jnp.float32),
        scratch_types=[
            pltpu.VMEM((64, 16), jnp.float32),
            pltpu.VMEM((128, 16), jnp.float32),
            pltpu.VMEM((4, 128), jnp.int32),
            pltpu.VMEM_SHARED((_NPAD, 16), jnp.float32),
        ],
        compiler_params=_SC_PARAMS)
    def k(dst_hbm, out_hbm, zbuf, ones_v, didx, dacc):
        c = lax.axis_index("c")
        s = lax.axis_index("s")

        @pl.loop(0, 64)
        def _(i):
            zbuf[i, :] = jnp.zeros((16,), jnp.float32)

        @pl.loop(0, 128)
        def _(i):
            ones_v[i, :] = jnp.ones((16,), jnp.float32)

        @pl.loop(0, _SPS, step=64)
        def _(t):
            pltpu.sync_copy(zbuf, dacc.at[pl.ds(s * _SPS + t, 64)])

        plsc.subcore_barrier()

        @pl.loop(0, _RPS_DEG, step=4)
        def _(r):
            row0 = (c * 16 + s) * _RPS_DEG + r
            pltpu.sync_copy(dst_hbm.at[pl.ds(row0, 4)], didx)
            for j in range(4):
                pltpu.sync_copy(ones_v, dacc.at[didx.at[j]], add=True)

        plsc.subcore_barrier()
        pltpu.sync_copy(dacc.at[pl.ds(s * _SPS, _SPS)],
                        out_hbm.at[pl.ds(c * _NPAD + s * _SPS, _SPS)])

    return k(dst2)


def _sc_conv(xsf, src2, dst2):
    @functools.partial(
        pl.kernel, mesh=_mesh(),
        out_type=jax.ShapeDtypeStruct((2 * _NPAD, 32), jnp.float32),
        scratch_types=[
            pltpu.VMEM((64, 32), jnp.float32),
            pltpu.VMEM((1, 64), jnp.int32),
            pltpu.VMEM((1, 64), jnp.int32),
            pltpu.VMEM((1, 64), jnp.int32),
            pltpu.VMEM((1, 64), jnp.int32),
            pltpu.VMEM((64, 128), jnp.float32),
            pltpu.VMEM((64, 128), jnp.float32),
            pltpu.VMEM((64, 32), jnp.float32),
            pltpu.VMEM((64, 32), jnp.float32),
            pltpu.VMEM_SHARED((_NPAD, 32), jnp.float32),
            pltpu.SemaphoreType.DMA,
            pltpu.SemaphoreType.DMA,
            pltpu.SemaphoreType.DMA,
            pltpu.SemaphoreType.DMA,
            pltpu.SemaphoreType.DMA,
            pltpu.SemaphoreType.DMA,
            pltpu.SemaphoreType.DMA,
            pltpu.SemaphoreType.DMA,
        ],
        compiler_params=_SC_PARAMS)
    def k(xs_hbm, src_hbm, dst_hbm, out_hbm, zbuf, sidxA, didxA, sidxB, didxB,
          rowsA, rowsB, r32A, r32B, acc, gA, gB, sA, sB, iA, iB, jA, jB):
        c = lax.axis_index("c")
        s = lax.axis_index("s")
        off = c * 32

        @pl.loop(0, 64)
        def _(i):
            zbuf[i, pl.ds(0, 16)] = jnp.zeros((16,), jnp.float32)
            zbuf[i, pl.ds(16, 16)] = jnp.zeros((16,), jnp.float32)

        @pl.loop(0, _SPS, step=64)
        def _(t):
            pltpu.sync_copy(zbuf, acc.at[pl.ds(s * _SPS + t, 64)])

        plsc.subcore_barrier()

        base = s * _RPS

        def phase(t, sidx, didx, rows, r32, gsem, ssem, isem, jsem):
            pltpu.make_async_copy(xs_hbm.at[sidx.at[0]], rows, gsem).wait()

            @pl.when(t + 2 < _RPS)
            def _():
                pltpu.async_copy(src_hbm.at[pl.ds(base + t + 2, 1)], sidx,
                                 isem)

            @pl.when(t > 1)
            def _():
                pltpu.make_async_copy(r32, acc.at[didx.at[0]], ssem).wait()

            pltpu.async_copy(dst_hbm.at[pl.ds(base + t, 1)], didx, jsem)

            @pl.loop(0, 64, step=8)
            def _(i0):
                for j in range(8):
                    r32[i0 + j, pl.ds(0, 16)] = rows[i0 + j, pl.ds(off, 16)]
                    r32[i0 + j, pl.ds(16, 16)] = rows[i0 + j,
                                                      pl.ds(off + 16, 16)]

            pltpu.make_async_copy(dst_hbm.at[pl.ds(base + t, 1)], didx,
                                  jsem).wait()
            pltpu.async_copy(r32, acc.at[didx.at[0]], ssem, add=True)

            @pl.when(t + 2 < _RPS)
            def _():
                pltpu.make_async_copy(src_hbm.at[pl.ds(base, 1)], sidx,
                                      isem).wait()
                pltpu.async_copy(xs_hbm.at[sidx.at[0]], rows, gsem)

        pltpu.sync_copy(src_hbm.at[pl.ds(base, 1)], sidxA)
        pltpu.sync_copy(src_hbm.at[pl.ds(base + 1, 1)], sidxB)
        pltpu.async_copy(xs_hbm.at[sidxA.at[0]], rowsA, gA)
        pltpu.async_copy(xs_hbm.at[sidxB.at[0]], rowsB, gB)

        @pl.loop(0, _RPS, step=2)
        def _(t):
            phase(t, sidxA, didxA, rowsA, r32A, gA, sA, iA, jA)
            phase(t + 1, sidxB, didxB, rowsB, r32B, gB, sB, iB, jB)

        pltpu.make_async_copy(r32A, acc.at[didxA.at[0]], sA).wait()
        pltpu.make_async_copy(r32B, acc.at[didxB.at[0]], sB).wait()

        plsc.subcore_barrier()
        pltpu.sync_copy(acc.at[pl.ds(s * _SPS, _SPS)],
                        out_hbm.at[pl.ds(c * _NPAD + s * _SPS, _SPS)])

    return k(xsf, src2, dst2)


_DOT = dict(preferred_element_type=jnp.float32, precision=lax.Precision.HIGHEST)


def _dinv_of(degp_blk):
    d16 = lax.rsqrt(degp_blk[0] + degp_blk[1] + 1.0)
    return jnp.concatenate([d16, d16, d16, d16], axis=1)


def _enc_body(x_ref, we_ref, be_ref, wc1_ref, o_ref):
    h = jnp.maximum(
        lax.dot_general(x_ref[...], we_ref[...], (((1,), (1,)), ((), ())), **_DOT)
        + be_ref[...], 0.0)
    o_ref[...] = lax.dot_general(h, wc1_ref[...], (((1,), (1,)), ((), ())), **_DOT)


def _tc_encode(x, W_enc, b_enc, Wc1):
    return pl.pallas_call(
        _enc_body,
        grid=(_G,),
        in_specs=[
            pl.BlockSpec((_BLK, 128), lambda i: (i, 0)),
            pl.BlockSpec((64, 128), lambda i: (0, 0)),
            pl.BlockSpec((1, 64), lambda i: (0, 0)),
            pl.BlockSpec((64, 64), lambda i: (0, 0)),
        ],
        out_specs=pl.BlockSpec((_BLK, 64), lambda i: (i, 0)),
        out_shape=jax.ShapeDtypeStruct((_N, 64), jnp.float32),
    )(x, W_enc, b_enc.reshape(1, 64), Wc1)


def _pad128(xs):
    return jnp.concatenate([xs, jnp.zeros((_BLK, 64), jnp.float32)], axis=1)


def _prep_body(degp_ref, xw_ref, o_ref):
    dinv = _dinv_of(degp_ref[...])
    o_ref[...] = _pad128(xw_ref[...] * dinv)


def _tc_prep(degp, xw1):
    return pl.pallas_call(
        _prep_body,
        grid=(_G,),
        in_specs=[
            pl.BlockSpec((2, _BLK, 16), lambda i: (0, i, 0)),
            pl.BlockSpec((_BLK, 64), lambda i: (i, 0)),
        ],
        out_specs=pl.BlockSpec((_BLK, 128), lambda i: (i, 0)),
        out_shape=jax.ShapeDtypeStruct((_N, 128), jnp.float32),
    )(degp, xw1)


def _mid_body(agg_ref, xs_ref, degp_ref, b_ref, w_ref, o_ref):
    dinv = _dinv_of(degp_ref[...])
    pre = jnp.concatenate(
        [agg_ref[0] + xs_ref[:, :32], agg_ref[1] + xs_ref[:, 32:64]], axis=1)
    h = jnp.maximum(pre * dinv + b_ref[...], 0.0)
    xw = lax.dot_general(h, w_ref[...], (((1,), (1,)), ((), ())), **_DOT)
    o_ref[...] = _pad128(xw * dinv)


def _tc_mid(agg, xs, degp, b_prev, W_next):
    return pl.pallas_call(
        _mid_body,
        grid=(_G,),
        in_specs=[
            pl.BlockSpec((2, _BLK, 32), lambda i: (0, i, 0)),
            pl.BlockSpec((_BLK, 128), lambda i: (i, 0)),
            pl.BlockSpec((2, _BLK, 16), lambda i: (0, i, 0)),
            pl.BlockSpec((1, 64), lambda i: (0, 0)),
            pl.BlockSpec((64, 64), lambda i: (0, 0)),
        ],
        out_specs=pl.BlockSpec((_BLK, 128), lambda i: (i, 0)),
        out_shape=jax.ShapeDtypeStruct((_N, 128), jnp.float32),
    )(agg, xs, degp, b_prev.reshape(1, 64), W_next)


def _heads_body(agg_ref, xs_ref, degp_ref, b_ref, wd1_ref, bd1_ref, wd2_ref,
                bd2_ref, wi1_ref, bi1_ref, wi2_ref, bi2_ref, od_ref, oi_ref):
    dinv = _dinv_of(degp_ref[...])
    pre = jnp.concatenate(
        [agg_ref[0] + xs_ref[:, :32], agg_ref[1] + xs_ref[:, 32:64]], axis=1)
    h = jnp.maximum(pre * dinv + b_ref[...], 0.0)
    hd = jnp.maximum(
        lax.dot_general(h, wd1_ref[...], (((1,), (1,)), ((), ())), **_DOT)
        + bd1_ref[...], 0.0)
    od_ref[...] = lax.dot_general(hd, wd2_ref[...], (((1,), (1,)), ((), ())),
                                  **_DOT) + bd2_ref[...]
    hi = jnp.maximum(
        lax.dot_general(h, wi1_ref[...], (((1,), (1,)), ((), ())), **_DOT)
        + bi1_ref[...], 0.0)
    oi_ref[...] = lax.dot_general(hi, wi2_ref[...], (((1,), (1,)), ((), ())),
                                  **_DOT) + bi2_ref[...]


def _pad_head(W2, b2):
    W2p = jnp.concatenate([W2, jnp.zeros((127, 32), jnp.float32)], axis=0)
    b2p = jnp.broadcast_to(b2.reshape(1, 1), (1, 128))
    return W2p, b2p


def _tc_heads(agg, xs, degp, bc3, Wd1, bd1, Wd2, bd2, Wi1, bi1, Wi2, bi2):
    full = lambda a, b: pl.BlockSpec((a, b), lambda i: (0, 0))
    return pl.pallas_call(
        _heads_body,
        grid=(_G,),
        in_specs=[
            pl.BlockSpec((2, _BLK, 32), lambda i: (0, i, 0)),
            pl.BlockSpec((_BLK, 128), lambda i: (i, 0)),
            pl.BlockSpec((2, _BLK, 16), lambda i: (0, i, 0)),
            full(1, 64),
            full(32, 64), full(1, 32), full(128, 32), full(1, 128),
            full(32, 64), full(1, 32), full(128, 32), full(1, 128),
        ],
        out_specs=[
            pl.BlockSpec((_BLK, 128), lambda i: (i, 0)),
            pl.BlockSpec((_BLK, 128), lambda i: (i, 0)),
        ],
        out_shape=[
            jax.ShapeDtypeStruct((_N, 128), jnp.float32),
            jax.ShapeDtypeStruct((_N, 128), jnp.float32),
        ],
    )(agg, xs, degp, bc3.reshape(1, 64),
      Wd1, bd1.reshape(1, 32), *_pad_head(Wd2, bd2),
      Wi1, bi1.reshape(1, 32), *_pad_head(Wi2, bi2))


def kernel(x, edge_index, W_enc, b_enc, Wc1, bc1, Wc2, bc2, Wc3, bc3,
           Wd1, bd1, Wd2, bd2, Wi1, bi1, Wi2, bi2):
    e = edge_index.shape[1]
    pad = _EPAD - e
    srcf = jnp.concatenate([edge_index[0], jnp.zeros((pad,), jnp.int32)])
    dstf = jnp.concatenate([edge_index[1], jnp.full((pad,), _N, jnp.int32)])
    src2 = srcf.reshape(_ROWS64, 64)
    dst2 = dstf.reshape(_ROWS64, 64)
    degp = _sc_deg(dstf.reshape(_ROWS, 128)).reshape(2, _NPAD, 16)
    xw1 = _tc_encode(x, W_enc, b_enc, Wc1)
    xs1 = _tc_prep(degp, xw1)
    agg1 = _sc_conv(xs1, src2, dst2).reshape(2, _NPAD, 32)
    xs2 = _tc_mid(agg1, xs1, degp, bc1, Wc2)
    agg2 = _sc_conv(xs2, src2, dst2).reshape(2, _NPAD, 32)
    xs3 = _tc_mid(agg2, xs2, degp, bc2, Wc3)
    agg3 = _sc_conv(xs3, src2, dst2).reshape(2, _NPAD, 32)
    demand, inventory = _tc_heads(agg3, xs3, degp, bc3,
                                  Wd1, bd1, Wd2, bd2, Wi1, bi1, Wi2, bi2)
    return (demand[:, :1], inventory[:, :1])


# speedup vs baseline: 11.0900x; 1.1820x over previous
import functools

import jax
import jax.numpy as jnp
from jax import lax
from jax.experimental import pallas as pl
from jax.experimental.pallas import tpu as pltpu
from jax.experimental.pallas import tpu_sc as plsc

_N = 50000
_H = 64
_NPAD = 50176
_SPS = _NPAD // 16
_EPAD = 802816
_ROWS = _EPAD // 128
_ROWS64 = _EPAD // 64
_RPS = _ROWS64 // 16
_RPS_DEG = _ROWS // 32
_BLK = 1000
_G = _N // _BLK


def _mesh():
    return plsc.VectorSubcoreMesh(core_axis_name="c", subcore_axis_name="s")


_SC_PARAMS = pltpu.CompilerParams(use_tc_tiling_on_sc=False)


def _sc_deg(dst2):
    @functools.partial(
        pl.kernel, mesh=_mesh(),
        out_type=jax.ShapeDtypeStruct((2 * _NPAD, 16), jnp.float32),
        scratch_types=[
            pltpu.VMEM((64, 16), jnp.float32),
            pltpu.VMEM((128, 16), jnp.float32),
            pltpu.VMEM((4, 128), jnp.int32),
            pltpu.VMEM_SHARED((_NPAD, 16), jnp.float32),
        ],
        compiler_params=_SC_PARAMS)
    def k(dst_hbm, out_hbm, zbuf, ones_v, didx, dacc):
        c = lax.axis_index("c")
        s = lax.axis_index("s")

        @pl.loop(0, 64)
        def _(i):
            zbuf[i, :] = jnp.zeros((16,), jnp.float32)

        @pl.loop(0, 128)
        def _(i):
            ones_v[i, :] = jnp.ones((16,), jnp.float32)

        @pl.loop(0, _SPS, step=64)
        def _(t):
            pltpu.sync_copy(zbuf, dacc.at[pl.ds(s * _SPS + t, 64)])

        plsc.subcore_barrier()

        @pl.loop(0, _RPS_DEG, step=4)
        def _(r):
            row0 = (c * 16 + s) * _RPS_DEG + r
            pltpu.sync_copy(dst_hbm.at[pl.ds(row0, 4)], didx)
            for j in range(4):
                pltpu.sync_copy(ones_v, dacc.at[didx.at[j]], add=True)

        plsc.subcore_barrier()
        pltpu.sync_copy(dacc.at[pl.ds(s * _SPS, _SPS)],
                        out_hbm.at[pl.ds(c * _NPAD + s * _SPS, _SPS)])

    return k(dst2)


def _sc_conv(xsf, src2, dst2):
    @functools.partial(
        pl.kernel, mesh=_mesh(),
        out_type=jax.ShapeDtypeStruct((2 * _NPAD, 32), jnp.float32),
        scratch_types=[
            pltpu.VMEM((64, 32), jnp.float32),
            pltpu.VMEM((1, 64), jnp.int32),
            pltpu.VMEM((1, 64), jnp.int32),
            pltpu.VMEM((1, 64), jnp.int32),
            pltpu.VMEM((1, 64), jnp.int32),
            pltpu.VMEM((64, 64), jnp.float32),
            pltpu.VMEM((64, 64), jnp.float32),
            pltpu.VMEM((64, 32), jnp.float32),
            pltpu.VMEM((64, 32), jnp.float32),
            pltpu.VMEM_SHARED((_NPAD, 32), jnp.float32),
            pltpu.SemaphoreType.DMA,
            pltpu.SemaphoreType.DMA,
            pltpu.SemaphoreType.DMA,
            pltpu.SemaphoreType.DMA,
            pltpu.SemaphoreType.DMA,
            pltpu.SemaphoreType.DMA,
            pltpu.SemaphoreType.DMA,
            pltpu.SemaphoreType.DMA,
        ],
        compiler_params=_SC_PARAMS)
    def k(xs_hbm, src_hbm, dst_hbm, out_hbm, zbuf, sidxA, didxA, sidxB, didxB,
          rowsA, rowsB, r32A, r32B, acc, gA, gB, sA, sB, iA, iB, jA, jB):
        c = lax.axis_index("c")
        s = lax.axis_index("s")
        off = c * 32

        @pl.loop(0, 64)
        def _(i):
            zbuf[i, pl.ds(0, 16)] = jnp.zeros((16,), jnp.float32)
            zbuf[i, pl.ds(16, 16)] = jnp.zeros((16,), jnp.float32)

        @pl.loop(0, _SPS, step=64)
        def _(t):
            pltpu.sync_copy(zbuf, acc.at[pl.ds(s * _SPS + t, 64)])

        plsc.subcore_barrier()

        base = s * _RPS

        def phase(t, sidx, didx, rows, r32, gsem, ssem, isem, jsem):
            pltpu.make_async_copy(xs_hbm.at[sidx.at[0]], rows, gsem).wait()

            @pl.when(t + 2 < _RPS)
            def _():
                pltpu.async_copy(src_hbm.at[pl.ds(base + t + 2, 1)], sidx,
                                 isem)

            @pl.when(t > 1)
            def _():
                pltpu.make_async_copy(r32, acc.at[didx.at[0]], ssem).wait()

            pltpu.async_copy(dst_hbm.at[pl.ds(base + t, 1)], didx, jsem)

            @pl.loop(0, 64, step=8)
            def _(i0):
                for j in range(8):
                    r32[i0 + j, pl.ds(0, 16)] = rows[i0 + j, pl.ds(off, 16)]
                    r32[i0 + j, pl.ds(16, 16)] = rows[i0 + j,
                                                      pl.ds(off + 16, 16)]

            pltpu.make_async_copy(dst_hbm.at[pl.ds(base + t, 1)], didx,
                                  jsem).wait()
            pltpu.async_copy(r32, acc.at[didx.at[0]], ssem, add=True)

            @pl.when(t + 2 < _RPS)
            def _():
                pltpu.make_async_copy(src_hbm.at[pl.ds(base, 1)], sidx,
                                      isem).wait()
                pltpu.async_copy(xs_hbm.at[sidx.at[0]], rows, gsem)

        pltpu.sync_copy(src_hbm.at[pl.ds(base, 1)], sidxA)
        pltpu.sync_copy(src_hbm.at[pl.ds(base + 1, 1)], sidxB)
        pltpu.async_copy(xs_hbm.at[sidxA.at[0]], rowsA, gA)
        pltpu.async_copy(xs_hbm.at[sidxB.at[0]], rowsB, gB)

        @pl.loop(0, _RPS, step=2)
        def _(t):
            phase(t, sidxA, didxA, rowsA, r32A, gA, sA, iA, jA)
            phase(t + 1, sidxB, didxB, rowsB, r32B, gB, sB, iB, jB)

        pltpu.make_async_copy(r32A, acc.at[didxA.at[0]], sA).wait()
        pltpu.make_async_copy(r32B, acc.at[didxB.at[0]], sB).wait()

        plsc.subcore_barrier()
        pltpu.sync_copy(acc.at[pl.ds(s * _SPS, _SPS)],
                        out_hbm.at[pl.ds(c * _NPAD + s * _SPS, _SPS)])

    return k(xsf, src2, dst2)


_DOT = dict(preferred_element_type=jnp.float32, precision=lax.Precision.HIGHEST)


def _dinv_of(degp_blk):
    d16 = lax.rsqrt(degp_blk[0] + degp_blk[1] + 1.0)
    return jnp.concatenate([d16, d16, d16, d16], axis=1)


def _enc_body(x_ref, we_ref, be_ref, wc1_ref, o_ref):
    h = jnp.maximum(
        lax.dot_general(x_ref[...], we_ref[...], (((1,), (1,)), ((), ())), **_DOT)
        + be_ref[...], 0.0)
    o_ref[...] = lax.dot_general(h, wc1_ref[...], (((1,), (1,)), ((), ())), **_DOT)


def _tc_encode(x, W_enc, b_enc, Wc1):
    return pl.pallas_call(
        _enc_body,
        grid=(_G,),
        in_specs=[
            pl.BlockSpec((_BLK, 128), lambda i: (i, 0)),
            pl.BlockSpec((64, 128), lambda i: (0, 0)),
            pl.BlockSpec((1, 64), lambda i: (0, 0)),
            pl.BlockSpec((64, 64), lambda i: (0, 0)),
        ],
        out_specs=pl.BlockSpec((_BLK, 64), lambda i: (i, 0)),
        out_shape=jax.ShapeDtypeStruct((_N, 64), jnp.float32),
    )(x, W_enc, b_enc.reshape(1, 64), Wc1)


def _prep_body(degp_ref, xw_ref, o_ref):
    dinv = _dinv_of(degp_ref[...])
    o_ref[...] = xw_ref[...] * dinv


def _tc_prep(degp, xw1):
    return pl.pallas_call(
        _prep_body,
        grid=(_G,),
        in_specs=[
            pl.BlockSpec((2, _BLK, 16), lambda i: (0, i, 0)),
            pl.BlockSpec((_BLK, 64), lambda i: (i, 0)),
        ],
        out_specs=pl.BlockSpec((_BLK, 64), lambda i: (i, 0)),
        out_shape=jax.ShapeDtypeStruct((_N, 64), jnp.float32),
    )(degp, xw1)


def _mid_body(agg_ref, xs_ref, degp_ref, b_ref, w_ref, o_ref):
    dinv = _dinv_of(degp_ref[...])
    pre = jnp.concatenate(
        [agg_ref[0] + xs_ref[:, :32], agg_ref[1] + xs_ref[:, 32:64]], axis=1)
    h = jnp.maximum(pre * dinv + b_ref[...], 0.0)
    xw = lax.dot_general(h, w_ref[...], (((1,), (1,)), ((), ())), **_DOT)
    o_ref[...] = xw * dinv


def _tc_mid(agg, xs, degp, b_prev, W_next):
    return pl.pallas_call(
        _mid_body,
        grid=(_G,),
        in_specs=[
            pl.BlockSpec((2, _BLK, 32), lambda i: (0, i, 0)),
            pl.BlockSpec((_BLK, 64), lambda i: (i, 0)),
            pl.BlockSpec((2, _BLK, 16), lambda i: (0, i, 0)),
            pl.BlockSpec((1, 64), lambda i: (0, 0)),
            pl.BlockSpec((64, 64), lambda i: (0, 0)),
        ],
        out_specs=pl.BlockSpec((_BLK, 64), lambda i: (i, 0)),
        out_shape=jax.ShapeDtypeStruct((_N, 64), jnp.float32),
    )(agg, xs, degp, b_prev.reshape(1, 64), W_next)


def _heads_body(agg_ref, xs_ref, degp_ref, b_ref, wd1_ref, bd1_ref, wd2_ref,
                bd2_ref, wi1_ref, bi1_ref, wi2_ref, bi2_ref, od_ref, oi_ref):
    dinv = _dinv_of(degp_ref[...])
    pre = jnp.concatenate(
        [agg_ref[0] + xs_ref[:, :32], agg_ref[1] + xs_ref[:, 32:64]], axis=1)
    h = jnp.maximum(pre * dinv + b_ref[...], 0.0)
    hd = jnp.maximum(
        lax.dot_general(h, wd1_ref[...], (((1,), (1,)), ((), ())), **_DOT)
        + bd1_ref[...], 0.0)
    od_ref[...] = lax.dot_general(hd, wd2_ref[...], (((1,), (1,)), ((), ())),
                                  **_DOT) + bd2_ref[...]
    hi = jnp.maximum(
        lax.dot_general(h, wi1_ref[...], (((1,), (1,)), ((), ())), **_DOT)
        + bi1_ref[...], 0.0)
    oi_ref[...] = lax.dot_general(hi, wi2_ref[...], (((1,), (1,)), ((), ())),
                                  **_DOT) + bi2_ref[...]


def _pad_head(W2, b2):
    W2p = jnp.concatenate([W2, jnp.zeros((127, 32), jnp.float32)], axis=0)
    b2p = jnp.broadcast_to(b2.reshape(1, 1), (1, 128))
    return W2p, b2p


def _tc_heads(agg, xs, degp, bc3, Wd1, bd1, Wd2, bd2, Wi1, bi1, Wi2, bi2):
    full = lambda a, b: pl.BlockSpec((a, b), lambda i: (0, 0))
    return pl.pallas_call(
        _heads_body,
        grid=(_G,),
        in_specs=[
            pl.BlockSpec((2, _BLK, 32), lambda i: (0, i, 0)),
            pl.BlockSpec((_BLK, 64), lambda i: (i, 0)),
            pl.BlockSpec((2, _BLK, 16), lambda i: (0, i, 0)),
            full(1, 64),
            full(32, 64), full(1, 32), full(128, 32), full(1, 128),
            full(32, 64), full(1, 32), full(128, 32), full(1, 128),
        ],
        out_specs=[
            pl.BlockSpec((_BLK, 128), lambda i: (i, 0)),
            pl.BlockSpec((_BLK, 128), lambda i: (i, 0)),
        ],
        out_shape=[
            jax.ShapeDtypeStruct((_N, 128), jnp.float32),
            jax.ShapeDtypeStruct((_N, 128), jnp.float32),
        ],
    )(agg, xs, degp, bc3.reshape(1, 64),
      Wd1, bd1.reshape(1, 32), *_pad_head(Wd2, bd2),
      Wi1, bi1.reshape(1, 32), *_pad_head(Wi2, bi2))


def kernel(x, edge_index, W_enc, b_enc, Wc1, bc1, Wc2, bc2, Wc3, bc3,
           Wd1, bd1, Wd2, bd2, Wi1, bi1, Wi2, bi2):
    e = edge_index.shape[1]
    pad = _EPAD - e
    srcf = jnp.concatenate([edge_index[0], jnp.zeros((pad,), jnp.int32)])
    dstf = jnp.concatenate([edge_index[1], jnp.full((pad,), _N, jnp.int32)])
    src2 = srcf.reshape(_ROWS64, 64)
    dst2 = dstf.reshape(_ROWS64, 64)
    degp = _sc_deg(dstf.reshape(_ROWS, 128)).reshape(2, _NPAD, 16)
    xw1 = _tc_encode(x, W_enc, b_enc, Wc1)
    xs1 = _tc_prep(degp, xw1)
    agg1 = _sc_conv(xs1, src2, dst2).reshape(2, _NPAD, 32)
    xs2 = _tc_mid(agg1, xs1, degp, bc1, Wc2)
    agg2 = _sc_conv(xs2, src2, dst2).reshape(2, _NPAD, 32)
    xs3 = _tc_mid(agg2, xs2, degp, bc2, Wc3)
    agg3 = _sc_conv(xs3, src2, dst2).reshape(2, _NPAD, 32)
    demand, inventory = _tc_heads(agg3, xs3, degp, bc3,
                                  Wd1, bd1, Wd2, bd2, Wi1, bi1, Wi2, bi2)
    return (demand[:, :1], inventory[:, :1])


# speedup vs baseline: 14.9005x; 1.3436x over previous
import functools

import jax
import jax.numpy as jnp
from jax import lax
from jax.experimental import pallas as pl
from jax.experimental.pallas import tpu as pltpu
from jax.experimental.pallas import tpu_sc as plsc

_N = 50000
_H = 64
_NPAD = 50176
_SPS = _NPAD // 16
_EPAD = 802816
_ROWS = _EPAD // 128
_RPS = _ROWS // 16
_RPS_DEG = _ROWS // 32
_BLK = 1000
_G = _N // _BLK


def _mesh():
    return plsc.VectorSubcoreMesh(core_axis_name="c", subcore_axis_name="s")


_SC_PARAMS = pltpu.CompilerParams(use_tc_tiling_on_sc=False)


def _sc_deg(dst2):
    @functools.partial(
        pl.kernel, mesh=_mesh(),
        out_type=jax.ShapeDtypeStruct((2 * _NPAD, 16), jnp.float32),
        scratch_types=[
            pltpu.VMEM((64, 16), jnp.float32),
            pltpu.VMEM((128, 16), jnp.float32),
            pltpu.VMEM((4, 128), jnp.int32),
            pltpu.VMEM_SHARED((_NPAD, 16), jnp.float32),
        ],
        compiler_params=_SC_PARAMS)
    def k(dst_hbm, out_hbm, zbuf, ones_v, didx, dacc):
        c = lax.axis_index("c")
        s = lax.axis_index("s")

        @pl.loop(0, 64)
        def _(i):
            zbuf[i, :] = jnp.zeros((16,), jnp.float32)

        @pl.loop(0, 128)
        def _(i):
            ones_v[i, :] = jnp.ones((16,), jnp.float32)

        @pl.loop(0, _SPS, step=64)
        def _(t):
            pltpu.sync_copy(zbuf, dacc.at[pl.ds(s * _SPS + t, 64)])

        plsc.subcore_barrier()

        @pl.loop(0, _RPS_DEG, step=4)
        def _(r):
            row0 = (c * 16 + s) * _RPS_DEG + r
            pltpu.sync_copy(dst_hbm.at[pl.ds(row0, 4)], didx)
            for j in range(4):
                pltpu.sync_copy(ones_v, dacc.at[didx.at[j]], add=True)

        plsc.subcore_barrier()
        pltpu.sync_copy(dacc.at[pl.ds(s * _SPS, _SPS)],
                        out_hbm.at[pl.ds(c * _NPAD + s * _SPS, _SPS)])

    return k(dst2)


def _sc_conv(xsf, src2, dst2):
    @functools.partial(
        pl.kernel, mesh=_mesh(),
        out_type=jax.ShapeDtypeStruct((2 * _NPAD, 32), jnp.float32),
        scratch_types=[
            pltpu.VMEM((64, 32), jnp.float32),
            pltpu.VMEM((1, 128), jnp.int32),
            pltpu.VMEM((1, 128), jnp.int32),
            pltpu.VMEM((1, 128), jnp.int32),
            pltpu.VMEM((1, 128), jnp.int32),
            pltpu.VMEM((128, 32), jnp.float32),
            pltpu.VMEM((128, 32), jnp.float32),
            pltpu.VMEM_SHARED((_NPAD, 32), jnp.float32),
            pltpu.SemaphoreType.DMA,
            pltpu.SemaphoreType.DMA,
            pltpu.SemaphoreType.DMA,
            pltpu.SemaphoreType.DMA,
            pltpu.SemaphoreType.DMA,
            pltpu.SemaphoreType.DMA,
            pltpu.SemaphoreType.DMA,
            pltpu.SemaphoreType.DMA,
        ],
        compiler_params=_SC_PARAMS)
    def k(xs_hbm, src_hbm, dst_hbm, out_hbm, zbuf, sidxA, didxA, sidxB, didxB,
          rowsA, rowsB, acc, gA, gB, sA, sB, iA, iB, jA, jB):
        c = lax.axis_index("c")
        s = lax.axis_index("s")

        @pl.loop(0, 64)
        def _(i):
            zbuf[i, pl.ds(0, 16)] = jnp.zeros((16,), jnp.float32)
            zbuf[i, pl.ds(16, 16)] = jnp.zeros((16,), jnp.float32)

        @pl.loop(0, _SPS, step=64)
        def _(t):
            pltpu.sync_copy(zbuf, acc.at[pl.ds(s * _SPS + t, 64)])

        plsc.subcore_barrier()

        base = c * _ROWS + s * _RPS
        dbase = s * _RPS

        def phase(t, this, other):
            sidxT, didxT, rowsT, gT, sT, iT, jT = this
            sidxO, didxO, rowsO, gO, sO, iO, jO = other
            pltpu.make_async_copy(xs_hbm.at[sidxT.at[0]], rowsT, gT).wait()

            @pl.when(t + 2 < _RPS)
            def _():
                pltpu.async_copy(src_hbm.at[pl.ds(base + t + 2, 1)], sidxT,
                                 iT)

            pltpu.make_async_copy(dst_hbm.at[pl.ds(dbase + t, 1)], didxT,
                                  jT).wait()
            pltpu.async_copy(rowsT, acc.at[didxT.at[0]], sT, add=True)

            @pl.when(t + 1 < _RPS)
            def _():
                @pl.when(t > 0)
                def _():
                    pltpu.make_async_copy(rowsO, acc.at[didxO.at[0]],
                                          sO).wait()
                pltpu.make_async_copy(src_hbm.at[pl.ds(base, 1)], sidxO,
                                      iO).wait()
                pltpu.async_copy(xs_hbm.at[sidxO.at[0]], rowsO, gO)
                pltpu.async_copy(dst_hbm.at[pl.ds(dbase + t + 1, 1)], didxO,
                                 jO)

        pltpu.sync_copy(src_hbm.at[pl.ds(base, 1)], sidxA)
        pltpu.async_copy(src_hbm.at[pl.ds(base + 1, 1)], sidxB, iB)
        pltpu.async_copy(xs_hbm.at[sidxA.at[0]], rowsA, gA)
        pltpu.async_copy(dst_hbm.at[pl.ds(dbase, 1)], didxA, jA)

        slotA = (sidxA, didxA, rowsA, gA, sA, iA, jA)
        slotB = (sidxB, didxB, rowsB, gB, sB, iB, jB)

        @pl.loop(0, _RPS, step=2)
        def _(t):
            phase(t, slotA, slotB)
            phase(t + 1, slotB, slotA)

        pltpu.make_async_copy(rowsA, acc.at[didxA.at[0]], sA).wait()
        pltpu.make_async_copy(rowsB, acc.at[didxB.at[0]], sB).wait()

        plsc.subcore_barrier()
        pltpu.sync_copy(acc.at[pl.ds(s * _SPS, _SPS)],
                        out_hbm.at[pl.ds(c * _NPAD + s * _SPS, _SPS)])

    return k(xsf, src2, dst2)


_DOT = dict(preferred_element_type=jnp.float32, precision=lax.Precision.HIGHEST)


def _dinv_of(degp_blk):
    d16 = lax.rsqrt(degp_blk[0] + degp_blk[1] + 1.0)
    return jnp.concatenate([d16, d16, d16, d16], axis=1)


def _enc_body(x_ref, we_ref, be_ref, wc1_ref, o_ref):
    h = jnp.maximum(
        lax.dot_general(x_ref[...], we_ref[...], (((1,), (1,)), ((), ())), **_DOT)
        + be_ref[...], 0.0)
    o_ref[...] = lax.dot_general(h, wc1_ref[...], (((1,), (1,)), ((), ())), **_DOT)


def _tc_encode(x, W_enc, b_enc, Wc1):
    return pl.pallas_call(
        _enc_body,
        grid=(_G,),
        in_specs=[
            pl.BlockSpec((_BLK, 128), lambda i: (i, 0)),
            pl.BlockSpec((64, 128), lambda i: (0, 0)),
            pl.BlockSpec((1, 64), lambda i: (0, 0)),
            pl.BlockSpec((64, 64), lambda i: (0, 0)),
        ],
        out_specs=pl.BlockSpec((_BLK, 64), lambda i: (i, 0)),
        out_shape=jax.ShapeDtypeStruct((_N, 64), jnp.float32),
    )(x, W_enc, b_enc.reshape(1, 64), Wc1)


def _prep_body(degp_ref, xw_ref, o_ref):
    dinv = _dinv_of(degp_ref[...])
    xsv = xw_ref[...] * dinv
    o_ref[0] = xsv[:, :32]
    o_ref[1] = xsv[:, 32:]


def _tc_prep(degp, xw1):
    return pl.pallas_call(
        _prep_body,
        grid=(_G,),
        in_specs=[
            pl.BlockSpec((2, _BLK, 16), lambda i: (0, i, 0)),
            pl.BlockSpec((_BLK, 64), lambda i: (i, 0)),
        ],
        out_specs=pl.BlockSpec((2, _BLK, 32), lambda i: (0, i, 0)),
        out_shape=jax.ShapeDtypeStruct((2, _N, 32), jnp.float32),
    )(degp, xw1)


def _mid_body(agg_ref, xs_ref, degp_ref, b_ref, w_ref, o_ref):
    dinv = _dinv_of(degp_ref[...])
    pre = jnp.concatenate(
        [agg_ref[0] + xs_ref[0], agg_ref[1] + xs_ref[1]], axis=1)
    h = jnp.maximum(pre * dinv + b_ref[...], 0.0)
    xw = lax.dot_general(h, w_ref[...], (((1,), (1,)), ((), ())), **_DOT)
    xsv = xw * dinv
    o_ref[0] = xsv[:, :32]
    o_ref[1] = xsv[:, 32:]


def _tc_mid(agg, xs, degp, b_prev, W_next):
    return pl.pallas_call(
        _mid_body,
        grid=(_G,),
        in_specs=[
            pl.BlockSpec((2, _BLK, 32), lambda i: (0, i, 0)),
            pl.BlockSpec((2, _BLK, 32), lambda i: (0, i, 0)),
            pl.BlockSpec((2, _BLK, 16), lambda i: (0, i, 0)),
            pl.BlockSpec((1, 64), lambda i: (0, 0)),
            pl.BlockSpec((64, 64), lambda i: (0, 0)),
        ],
        out_specs=pl.BlockSpec((2, _BLK, 32), lambda i: (0, i, 0)),
        out_shape=jax.ShapeDtypeStruct((2, _N, 32), jnp.float32),
    )(agg, xs, degp, b_prev.reshape(1, 64), W_next)


def _heads_body(agg_ref, xs_ref, degp_ref, b_ref, wd1_ref, bd1_ref, wd2_ref,
                bd2_ref, wi1_ref, bi1_ref, wi2_ref, bi2_ref, od_ref, oi_ref):
    dinv = _dinv_of(degp_ref[...])
    pre = jnp.concatenate(
        [agg_ref[0] + xs_ref[0], agg_ref[1] + xs_ref[1]], axis=1)
    h = jnp.maximum(pre * dinv + b_ref[...], 0.0)
    hd = jnp.maximum(
        lax.dot_general(h, wd1_ref[...], (((1,), (1,)), ((), ())), **_DOT)
        + bd1_ref[...], 0.0)
    od_ref[...] = lax.dot_general(hd, wd2_ref[...], (((1,), (1,)), ((), ())),
                                  **_DOT) + bd2_ref[...]
    hi = jnp.maximum(
        lax.dot_general(h, wi1_ref[...], (((1,), (1,)), ((), ())), **_DOT)
        + bi1_ref[...], 0.0)
    oi_ref[...] = lax.dot_general(hi, wi2_ref[...], (((1,), (1,)), ((), ())),
                                  **_DOT) + bi2_ref[...]


def _pad_head(W2, b2):
    W2p = jnp.concatenate([W2, jnp.zeros((127, 32), jnp.float32)], axis=0)
    b2p = jnp.broadcast_to(b2.reshape(1, 1), (1, 128))
    return W2p, b2p


def _tc_heads(agg, xs, degp, bc3, Wd1, bd1, Wd2, bd2, Wi1, bi1, Wi2, bi2):
    full = lambda a, b: pl.BlockSpec((a, b), lambda i: (0, 0))
    return pl.pallas_call(
        _heads_body,
        grid=(_G,),
        in_specs=[
            pl.BlockSpec((2, _BLK, 32), lambda i: (0, i, 0)),
            pl.BlockSpec((2, _BLK, 32), lambda i: (0, i, 0)),
            pl.BlockSpec((2, _BLK, 16), lambda i: (0, i, 0)),
            full(1, 64),
            full(32, 64), full(1, 32), full(128, 32), full(1, 128),
            full(32, 64), full(1, 32), full(128, 32), full(1, 128),
        ],
        out_specs=[
            pl.BlockSpec((_BLK, 128), lambda i: (i, 0)),
            pl.BlockSpec((_BLK, 128), lambda i: (i, 0)),
        ],
        out_shape=[
            jax.ShapeDtypeStruct((_N, 128), jnp.float32),
            jax.ShapeDtypeStruct((_N, 128), jnp.float32),
        ],
    )(agg, xs, degp, bc3.reshape(1, 64),
      Wd1, bd1.reshape(1, 32), *_pad_head(Wd2, bd2),
      Wi1, bi1.reshape(1, 32), *_pad_head(Wi2, bi2))


def kernel(x, edge_index, W_enc, b_enc, Wc1, bc1, Wc2, bc2, Wc3, bc3,
           Wd1, bd1, Wd2, bd2, Wi1, bi1, Wi2, bi2):
    e = edge_index.shape[1]
    pad = _EPAD - e
    srcf = jnp.concatenate([edge_index[0], jnp.zeros((pad,), jnp.int32)])
    dstf = jnp.concatenate([edge_index[1], jnp.full((pad,), _N, jnp.int32)])
    src2 = jnp.concatenate([srcf, srcf + _N]).reshape(2 * _ROWS, 128)
    dst2 = dstf.reshape(_ROWS, 128)
    degp = _sc_deg(dst2).reshape(2, _NPAD, 16)
    xw1 = _tc_encode(x, W_enc, b_enc, Wc1)
    xs1 = _tc_prep(degp, xw1)
    agg1 = _sc_conv(xs1.reshape(2 * _N, 32), src2, dst2).reshape(2, _NPAD, 32)
    xs2 = _tc_mid(agg1, xs1, degp, bc1, Wc2)
    agg2 = _sc_conv(xs2.reshape(2 * _N, 32), src2, dst2).reshape(2, _NPAD, 32)
    xs3 = _tc_mid(agg2, xs2, degp, bc2, Wc3)
    agg3 = _sc_conv(xs3.reshape(2 * _N, 32), src2, dst2).reshape(2, _NPAD, 32)
    demand, inventory = _tc_heads(agg3, xs3, degp, bc3,
                                  Wd1, bd1, Wd2, bd2, Wi1, bi1, Wi2, bi2)
    return (demand[:, :1], inventory[:, :1])
